# baseline (speedup 1.0000x reference)
import functools

import jax
import jax.numpy as jnp
from jax import lax
from jax.experimental import pallas as pl
from jax.experimental.pallas import tpu as pltpu
from jax.experimental.pallas import tpu_sc as plsc

NBUF = 2


def _build_lookup(b, l, emb):
    info = plsc.get_sparse_core_info()
    nc, ns = info.num_cores, info.num_subcores
    nw = nc * ns
    rows_w = b // nw

    mesh = plsc.VectorSubcoreMesh(core_axis_name="c", subcore_axis_name="s")

    @functools.partial(
        pl.kernel,
        mesh=mesh,
        out_type=jax.ShapeDtypeStruct((b, l, emb), jnp.float32),
        scratch_types=[
            pltpu.VMEM((rows_w, l), jnp.int32),
            pltpu.VMEM((l, emb), jnp.float32),
            pltpu.VMEM((l, emb), jnp.float32),
            pltpu.SemaphoreType.DMA,
            pltpu.SemaphoreType.DMA,
        ],
        compiler_params=pltpu.CompilerParams(use_tc_tiling_on_sc=False),
    )
    def lookup(x_hbm, table_hbm, out_hbm, idx_v, buf0, buf1, sem0, sem1):
        wid = lax.axis_index("s") * nc + lax.axis_index("c")
        base = wid * rows_w
        bufs = (buf0, buf1)
        sems = (sem0, sem1)

        def fire(r, buf, sem):
            pltpu.async_copy(table_hbm.at[idx_v.at[r]], buf, sem)

        def drain(r, buf, sem):
            pltpu.make_async_copy(out_hbm.at[base + r], buf, sem).wait()

        pltpu.sync_copy(x_hbm.at[pl.ds(base, rows_w)], idx_v)

        fire(0, buf0, sem0)

        def outer(i, carry):
            og = i * NBUF
            for p in range(NBUF):
                r = og + p
                np_ = 1 - p

                @pl.when(r + 1 < rows_w)
                def _():
                    fire(r + 1, bufs[np_], sems[np_])

                drain(r, bufs[p], sems[p])
                pltpu.sync_copy(bufs[p], out_hbm.at[base + r])
            return carry

        lax.fori_loop(0, rows_w // NBUF, outer, 0)

    return lookup


def kernel(x, table):
    b, l = x.shape
    emb = table.shape[1]
    return _build_lookup(b, l, emb)(x, table)


# speedup vs baseline: 1.3300x; 1.3300x over previous
import functools

import jax
import jax.numpy as jnp
from jax import lax
from jax.experimental import pallas as pl
from jax.experimental.pallas import tpu as pltpu
from jax.experimental.pallas import tpu_sc as plsc

NBUF = 2
PAD = 128


def _build_lookup(b, l, emb, vocab):
    info = plsc.get_sparse_core_info()
    nc, ns = info.num_cores, info.num_subcores
    nw = nc * ns
    rows_w = b // nw

    mesh = plsc.VectorSubcoreMesh(core_axis_name="c", subcore_axis_name="s")

    @functools.partial(
        pl.kernel,
        mesh=mesh,
        out_type=jax.ShapeDtypeStruct((b * l, PAD), jnp.float32),
        scratch_types=[
            pltpu.VMEM((rows_w, l), jnp.int32),
            pltpu.VMEM((l, emb), jnp.float32),
            pltpu.VMEM((l, emb), jnp.float32),
            pltpu.SemaphoreType.DMA,
            pltpu.SemaphoreType.DMA,
        ],
        compiler_params=pltpu.CompilerParams(use_tc_tiling_on_sc=False),
    )
    def lookup(x_hbm, table_hbm, out_hbm, idx_v, buf0, buf1, sem0, sem1):
        wid = lax.axis_index("s") * nc + lax.axis_index("c")
        base = wid * rows_w
        bufs = (buf0, buf1)
        sems = (sem0, sem1)

        def fire(r, buf, sem):
            pltpu.async_copy(table_hbm.at[idx_v.at[r]], buf, sem)

        def out_slice(r):
            return out_hbm.at[pl.ds((base + r) * l, l), pl.ds(0, emb)]

        def drain(r, buf, sem):
            pltpu.make_async_copy(out_slice(r), buf, sem).wait()

        pltpu.sync_copy(x_hbm.at[pl.ds(base, rows_w)], idx_v)

        fire(0, buf0, sem0)

        def outer(i, carry):
            og = i * NBUF
            for p in range(NBUF):
                r = og + p
                np_ = 1 - p

                @pl.when(r + 1 < rows_w)
                def _():
                    fire(r + 1, bufs[np_], sems[np_])

                drain(r, bufs[p], sems[p])
                pltpu.sync_copy(bufs[p], out_slice(r))
            return carry

        lax.fori_loop(0, rows_w // NBUF, outer, 0)

    return lookup


def kernel(x, table):
    b, l = x.shape
    vocab, emb = table.shape
    out = _build_lookup(b, l, emb, vocab)(x, table)
    return out[:, :emb].reshape(b, l, emb)


# speedup vs baseline: 1.3338x; 1.0029x over previous
import functools

import jax
import jax.numpy as jnp
from jax import lax
from jax.experimental import pallas as pl
from jax.experimental.pallas import tpu as pltpu
from jax.experimental.pallas import tpu_sc as plsc

NBUF = 2
PAD = 128


def _build_lookup(b, l, emb):
    info = plsc.get_sparse_core_info()
    nc, ns = info.num_cores, info.num_subcores
    nw = nc * ns
    rows_w = b // nw

    mesh = plsc.VectorSubcoreMesh(core_axis_name="c", subcore_axis_name="s")

    @functools.partial(
        pl.kernel,
        mesh=mesh,
        out_type=jax.ShapeDtypeStruct((b * l, PAD), jnp.float32),
        scratch_types=[
            pltpu.VMEM((rows_w, l), jnp.int32),
            pltpu.VMEM((l, emb), jnp.float32),
            pltpu.VMEM((l, emb), jnp.float32),
            pltpu.SemaphoreType.DMA,
            pltpu.SemaphoreType.DMA,
        ],
        compiler_params=pltpu.CompilerParams(use_tc_tiling_on_sc=False),
    )
    def lookup(x_hbm, table_hbm, out_hbm, idx_v, buf0, buf1, sem0, sem1):
        wid = lax.axis_index("s") * nc + lax.axis_index("c")
        base = wid * rows_w
        bufs = (buf0, buf1)
        sems = (sem0, sem1)

        def fire(r, buf, sem):
            pltpu.async_copy(table_hbm.at[idx_v.at[r]], buf, sem)

        def out_slice(r):
            return out_hbm.at[pl.ds((base + r) * l, l), pl.ds(0, emb)]

        def drain(r, buf, sem):
            pltpu.make_async_copy(out_slice(r), buf, sem).wait()

        pltpu.sync_copy(x_hbm.at[pl.ds(base, rows_w)], idx_v)

        fire(0, buf0, sem0)

        def outer(i, carry):
            og = i * NBUF
            for p in range(NBUF):
                r = og + p
                np_ = 1 - p

                @pl.when(r + 1 < rows_w)
                def _():
                    fire(r + 1, bufs[np_], sems[np_])

                drain(r, bufs[p], sems[p])
                pltpu.sync_copy(bufs[p], out_slice(r))
            return carry

        lax.fori_loop(0, rows_w // NBUF, outer, 0)

    return lookup


def kernel(x, table):
    b, l = x.shape
    emb = table.shape[1]
    out = _build_lookup(b, l, emb)(x, table)
    return out[:, :emb].reshape(b, l, emb)


# speedup vs baseline: 1.3340x; 1.0001x over previous
import functools

import jax
import jax.numpy as jnp
from jax import lax
from jax.experimental import pallas as pl
from jax.experimental.pallas import tpu as pltpu
from jax.experimental.pallas import tpu_sc as plsc

NBUF = 2
GRP = 2
PAD = 128


def _build_lookup(b, l, emb):
    info = plsc.get_sparse_core_info()
    nc, ns = info.num_cores, info.num_subcores
    nw = nc * ns
    rows_w = b // nw
    stages = rows_w // GRP

    mesh = plsc.VectorSubcoreMesh(core_axis_name="c", subcore_axis_name="s")

    @functools.partial(
        pl.kernel,
        mesh=mesh,
        out_type=jax.ShapeDtypeStruct((b * l, PAD), jnp.float32),
        scratch_types=[
            pltpu.VMEM((stages, GRP * l), jnp.int32),
            pltpu.VMEM((GRP * l, emb), jnp.float32),
            pltpu.VMEM((GRP * l, emb), jnp.float32),
            pltpu.SemaphoreType.DMA,
            pltpu.SemaphoreType.DMA,
        ],
        compiler_params=pltpu.CompilerParams(use_tc_tiling_on_sc=False),
    )
    def lookup(x_hbm, table_hbm, out_hbm, idx_v, buf0, buf1, sem0, sem1):
        wid = lax.axis_index("s") * nc + lax.axis_index("c")
        base = wid * rows_w
        bufs = (buf0, buf1)
        sems = (sem0, sem1)

        def fire(r, buf, sem):
            pltpu.async_copy(table_hbm.at[idx_v.at[r]], buf, sem)

        def out_slice(r):
            return out_hbm.at[pl.ds((base + r * GRP) * l, GRP * l), pl.ds(0, emb)]

        def drain(r, buf, sem):
            pltpu.make_async_copy(out_slice(r), buf, sem).wait()

        pltpu.sync_copy(x_hbm.at[pl.ds(wid * stages, stages)], idx_v)

        fire(0, buf0, sem0)

        def outer(i, carry):
            og = i * NBUF
            for p in range(NBUF):
                r = og + p
                np_ = 1 - p

                @pl.when(r + 1 < stages)
                def _():
                    fire(r + 1, bufs[np_], sems[np_])

                drain(r, bufs[p], sems[p])
                pltpu.sync_copy(bufs[p], out_slice(r))
            return carry

        lax.fori_loop(0, stages // NBUF, outer, 0)

    return lookup


def kernel(x, table):
    b, l = x.shape
    emb = table.shape[1]
    xg = x.reshape(b // GRP, GRP * l)
    out = _build_lookup(b, l, emb)(xg, table)
    return out[:, :emb].reshape(b, l, emb)
